# baseline (speedup 1.0000x reference)
import jax
import jax.numpy as jnp
from jax.experimental import pallas as pl

N = 10000
E = 320000
D = 128

_BN = 2000
_BE = 3200


def _node_body(h_ref, s1b_ref, s1a_ref, b1a_ref, w1a_ref, p_ref, g_ref):
    h = h_ref[...]
    p = h * s1b_ref[...]
    a = jnp.maximum(h * s1a_ref[...] + b1a_ref[...], 0.0)
    a = jnp.dot(a, w1a_ref[...], preferred_element_type=jnp.float32)
    p_ref[...] = p.astype(jnp.bfloat16)
    g_ref[...] = jnp.concatenate([p, a], axis=1).astype(jnp.bfloat16)


def _edge_body(ps_ref, gd_ref, b1_ref, w1_ref, s2_ref, b2_ref, w2_ref, o_ref):
    gd = gd_ref[...]
    d = ps_ref[...].astype(jnp.float32) - gd[:, :D].astype(jnp.float32)
    t = jnp.maximum(d + b1_ref[...], 0.0)
    t = jnp.dot(t.astype(jnp.bfloat16), w1_ref[...],
                preferred_element_type=jnp.float32)
    z = gd[:, D:].astype(jnp.float32) + t
    z = jnp.maximum(z * s2_ref[...] + b2_ref[...], 0.0)
    u = jnp.dot(z, w2_ref[...], preferred_element_type=jnp.float32)
    o_ref[...] = u


def _row_spec(bm, d):
    return pl.BlockSpec((bm, d), lambda i: (i, 0))


def _full_spec(shape):
    return pl.BlockSpec(shape, lambda i: (0,) * len(shape))


@jax.jit
def _node_tables(h, s1b, s1a, b1a, w1at):
    return pl.pallas_call(
        _node_body,
        grid=(N // _BN,),
        in_specs=[_row_spec(_BN, D), _full_spec((1, D)), _full_spec((1, D)),
                  _full_spec((1, D)), _full_spec((D, D))],
        out_specs=[_row_spec(_BN, D), _row_spec(_BN, 2 * D)],
        out_shape=[jax.ShapeDtypeStruct((N, D), jnp.bfloat16),
                   jax.ShapeDtypeStruct((N, 2 * D), jnp.bfloat16)],
    )(h, s1b, s1a, b1a, w1at)


@jax.jit
def _edge_mlp(ps, gd, b1b, w1bt, s2, b2, w2t):
    return pl.pallas_call(
        _edge_body,
        grid=(E // _BE,),
        in_specs=[_row_spec(_BE, D), _row_spec(_BE, 2 * D),
                  _full_spec((1, D)), _full_spec((D, D)),
                  _full_spec((1, D)), _full_spec((1, D)), _full_spec((D, D))],
        out_specs=_row_spec(_BE, D),
        out_shape=jax.ShapeDtypeStruct((E, D), jnp.float32),
    )(ps, gd, b1b, w1bt, s2, b2, w2t)


def kernel(x, joint_edge_index, ctx_size, bn1_g, bn1_b, W1, bn2_g, bn2_b, W2,
           fc_w, fc_b):
    inv = 1.0 / jnp.sqrt(1.0 + 1e-5)
    ei_s = joint_edge_index[0]
    ei_t = joint_edge_index[1]
    neg_inf = jnp.float32(-jnp.inf)

    def conv(h, ei, i):
        s1 = bn1_g[i] * inv
        b1 = bn1_b[i]
        s1a, s1b = s1[:D].reshape(1, D), s1[D:].reshape(1, D)
        b1a, b1b = b1[:D].reshape(1, D), b1[D:].reshape(1, D)
        w1t = W1[i].T
        w1at = w1t[:D]
        w1bt = w1t[D:].astype(jnp.bfloat16)
        s2 = (bn2_g[i] * inv).reshape(1, D)
        b2 = bn2_b[i].reshape(1, D)
        w2t = W2[i].T
        src, dst = ei[0], ei[1]

        p, g = _node_tables(h, s1b, s1a, b1a, w1at)
        ps = p[src]
        gd = g[dst]
        u = _edge_mlp(ps, gd, b1b, w1bt, s2, b2, w2t)
        out = jax.ops.segment_max(u, dst, num_segments=N)
        return jnp.where(jnp.isfinite(out), out, 0.0)

    g1s = conv(x, ei_s, 0)
    g1st = conv(g1s, ei_t, 1)
    g2s = conv(g1st, ei_s, 2)
    g2st = conv(g2s, ei_t, 3) + g1st
    g3s = conv(g2st, ei_s, 4)
    g3st = conv(g3s, ei_t, 5) + g2st
    g4s = conv(g3st, ei_s, 6)
    g4st = conv(g4s, ei_t, 7) + g3st
    return g4st @ fc_w.T + fc_b


# speedup vs baseline: 1.0778x; 1.0778x over previous
import jax
import jax.numpy as jnp
from jax.experimental import pallas as pl
from jax.experimental.pallas import tpu as pltpu

N = 10000
E = 320000
D = 128

_BN = 2000
_BE = 3200
_SHIFTS = (1, 2, 4, 8, 16, 32, 64, 128, 256, 512, 1024, 2048)


def _node_body(h_ref, s1b_ref, s1a_ref, b1a_ref, w1a_ref, p_ref, g_ref):
    h = h_ref[...]
    p = h * s1b_ref[...]
    a = jnp.maximum(h * s1a_ref[...] + b1a_ref[...], 0.0)
    a = jnp.dot(a, w1a_ref[...], preferred_element_type=jnp.float32)
    p_ref[...] = p
    g_ref[...] = jnp.concatenate([p, a], axis=1)


def _edge_body(ps_ref, gd_ref, fl_ref, b1_ref, w1_ref, s2_ref, b2_ref,
               w2_ref, o_ref, carry_ref):
    gd = gd_ref[...]
    d = ps_ref[...] - gd[:, :D]
    t = jnp.maximum(d + b1_ref[...], 0.0)
    t = jnp.dot(t, w1_ref[...], preferred_element_type=jnp.float32)
    z = gd[:, D:] + t
    z = jnp.maximum(z * s2_ref[...] + b2_ref[...], 0.0)
    u = jnp.dot(z, w2_ref[...], preferred_element_type=jnp.float32)

    @pl.when(pl.program_id(0) == 0)
    def _():
        carry_ref[...] = jnp.full((8, D), -jnp.inf, jnp.float32)

    f = fl_ref[...].astype(jnp.float32)
    carry = carry_ref[0:1, :]
    v0 = jnp.where(f[0:1] > 0.5, u[0:1], jnp.maximum(u[0:1], carry))
    v = jnp.concatenate([v0, u[1:]], axis=0)
    neg = jnp.float32(-jnp.inf)
    for s in _SHIFTS:
        vs = jnp.concatenate([jnp.full((s, D), neg, jnp.float32), v[:-s]],
                             axis=0)
        fs = jnp.concatenate([jnp.zeros((s, D), jnp.float32), f[:-s]],
                             axis=0)
        v = jnp.where(f > 0.5, v, jnp.maximum(v, vs))
        f = jnp.maximum(f, fs)
    carry_ref[0:1, :] = v[-1:]
    o_ref[...] = v


def _row_spec(bm, d):
    return pl.BlockSpec((bm, d), lambda i: (i, 0))


def _full_spec(shape):
    return pl.BlockSpec(shape, lambda i: (0,) * len(shape))


@jax.jit
def _node_tables(h, s1b, s1a, b1a, w1at):
    return pl.pallas_call(
        _node_body,
        grid=(N // _BN,),
        in_specs=[_row_spec(_BN, D), _full_spec((1, D)), _full_spec((1, D)),
                  _full_spec((1, D)), _full_spec((D, D))],
        out_specs=[_row_spec(_BN, D), _row_spec(_BN, 2 * D)],
        out_shape=[jax.ShapeDtypeStruct((N, D), jnp.float32),
                   jax.ShapeDtypeStruct((N, 2 * D), jnp.float32)],
    )(h, s1b, s1a, b1a, w1at)


@jax.jit
def _edge_mlp_scan(ps, gd, fl, b1b, w1bt, s2, b2, w2t):
    return pl.pallas_call(
        _edge_body,
        grid=(E // _BE,),
        in_specs=[_row_spec(_BE, D), _row_spec(_BE, 2 * D), _row_spec(_BE, D),
                  _full_spec((1, D)), _full_spec((D, D)),
                  _full_spec((1, D)), _full_spec((1, D)), _full_spec((D, D))],
        out_specs=_row_spec(_BE, D),
        out_shape=jax.ShapeDtypeStruct((E, D), jnp.float32),
        scratch_shapes=[pltpu.VMEM((8, D), jnp.float32)],
        compiler_params=pltpu.CompilerParams(
            dimension_semantics=("arbitrary",)),
    )(ps, gd, fl, b1b, w1bt, s2, b2, w2t)


def kernel(x, joint_edge_index, ctx_size, bn1_g, bn1_b, W1, bn2_g, bn2_b, W2,
           fc_w, fc_b):
    inv = 1.0 / jnp.sqrt(1.0 + 1e-5)

    def prep(ei):
        src, dst = ei[0], ei[1]
        perm = jnp.argsort(dst)
        dst_s = dst[perm]
        src_s = src[perm]
        first = jnp.concatenate(
            [jnp.ones((1,), jnp.bool_), dst_s[1:] != dst_s[:-1]])
        flags = jnp.broadcast_to(first[:, None], (E, D)).astype(jnp.bfloat16)
        nodes = jnp.arange(N, dtype=dst_s.dtype)
        right = jnp.searchsorted(dst_s, nodes, side='right')
        left = jnp.searchsorted(dst_s, nodes, side='left')
        last_pos = jnp.maximum(right - 1, 0)
        has = right > left
        return src_s, dst_s, flags, last_pos, has

    prep_s = prep(joint_edge_index[0])
    prep_t = prep(joint_edge_index[1])

    def conv(h, pr, i):
        src_s, dst_s, flags, last_pos, has = pr
        s1 = bn1_g[i] * inv
        b1 = bn1_b[i]
        s1a, s1b = s1[:D].reshape(1, D), s1[D:].reshape(1, D)
        b1a, b1b = b1[:D].reshape(1, D), b1[D:].reshape(1, D)
        w1t = W1[i].T
        w1at = w1t[:D]
        w1bt = w1t[D:]
        s2 = (bn2_g[i] * inv).reshape(1, D)
        b2 = bn2_b[i].reshape(1, D)
        w2t = W2[i].T

        p, g = _node_tables(h, s1b, s1a, b1a, w1at)
        ps = p[src_s]
        gd = g[dst_s]
        u_scan = _edge_mlp_scan(ps, gd, flags, b1b, w1bt, s2, b2, w2t)
        out = u_scan[last_pos]
        return jnp.where(has[:, None], out, 0.0)

    g1s = conv(x, prep_s, 0)
    g1st = conv(g1s, prep_t, 1)
    g2s = conv(g1st, prep_s, 2)
    g2st = conv(g2s, prep_t, 3) + g1st
    g3s = conv(g2st, prep_s, 4)
    g3st = conv(g3s, prep_t, 5) + g2st
    g4s = conv(g3st, prep_s, 6)
    g4st = conv(g4s, prep_t, 7) + g3st
    return g4st @ fc_w.T + fc_b
